# Initial kernel scaffold; baseline (speedup 1.0000x reference)
import jax
import jax.numpy as jnp
from jax.experimental import pallas as pl


def kernel(pos, x, params, batch_vec):
    raise NotImplementedError("write your pallas kernel here")


# speedup vs baseline: 3.6787x; 3.6787x over previous
import functools

import jax
import jax.numpy as jnp
from jax.experimental import pallas as pl

_K = 16
_BN_EPS = 1e-6


def _lrelu(v):
    return jnp.where(v >= 0, v, 0.2 * v)


def _mm(a, b):
    return jax.lax.dot(a.astype(jnp.bfloat16), b.astype(jnp.bfloat16),
                       preferred_element_type=jnp.float32)


def _bn_t(y, gamma, beta):
    m = jnp.mean(y, axis=1, keepdims=True)
    v = jnp.mean((y - m) ** 2, axis=1, keepdims=True)
    yn = (y - m) / jnp.sqrt(v + _BN_EPS)
    return yn * gamma + beta


def _dense_body(x_ref, w_ref, b_ref, g_ref, bt_ref, o_ref, *, norm, act):
    y = _mm(w_ref[...], x_ref[...]) + b_ref[...]
    if norm:
        y = _bn_t(y, g_ref[...], bt_ref[...])
    if act:
        y = _lrelu(y)
    o_ref[...] = y


def _dense_res_body(x_ref, w_ref, b_ref, g_ref, bt_ref, r_ref, o_ref):
    y = _mm(w_ref[...], x_ref[...]) + b_ref[...]
    y = _bn_t(y, g_ref[...], bt_ref[...])
    o_ref[...] = _lrelu(y + r_ref[...])


def _dense_t(xt, layer, *, act=True, res=None):
    n = xt.shape[1]
    dout = layer['W'].shape[1]
    norm = 'gamma' in layer
    wt = layer['W'].T
    b = layer['b'].reshape(dout, 1)
    g = layer['gamma'].reshape(dout, 1) if norm else b
    bt = layer['beta'].reshape(dout, 1) if norm else b
    out_shape = jax.ShapeDtypeStruct((dout, n), jnp.float32)
    if res is None:
        body = functools.partial(_dense_body, norm=norm, act=act)
        return pl.pallas_call(body, out_shape=out_shape)(xt, wt, b, g, bt)
    return pl.pallas_call(_dense_res_body, out_shape=out_shape)(
        xt, wt, b, g, bt, res)


def _smlp_t(layers, xt, act=True):
    for l in layers:
        xt = _dense_t(xt, l, act=act)
    return xt


def _knn_body(q_ref, st_ref, o_ref, *, k, ns):
    qc = q_ref[...]
    st = st_ref[...]
    qn = jnp.sum(qc * qc, axis=1, keepdims=True)
    sn = jnp.sum(st * st, axis=0, keepdims=True)
    d = qn - 2.0 * _mm(qc, st) + sn
    tq = d.shape[0]
    col = jax.lax.broadcasted_iota(jnp.int32, (tq, ns), 1)
    cols = []
    big = jnp.float32(3.0e38)
    for _ in range(k):
        m = jnp.min(d, axis=1, keepdims=True)
        idx = jnp.min(jnp.where(d <= m, col, ns), axis=1)
        cols.append(idx[:, None])
        d = jnp.where(col == idx[:, None], big, d)
    o_ref[...] = jnp.concatenate(cols, axis=1) if k > 1 else cols[0]


def _knn(q, s, k):
    nq, ns = q.shape[0], s.shape[0]
    tq = min(128, -(-nq // 8) * 8)
    nqp = -(-nq // tq) * tq
    if nqp != nq:
        q = jnp.pad(q, ((0, nqp - nq), (0, 0)))
    st = s.T
    body = functools.partial(_knn_body, k=k, ns=ns)
    out = pl.pallas_call(
        body,
        grid=(nqp // tq,),
        in_specs=[pl.BlockSpec((tq, 3), lambda i: (i, 0)),
                  pl.BlockSpec((3, ns), lambda i: (0, 0))],
        out_specs=pl.BlockSpec((tq, k), lambda i: (i, 0)),
        out_shape=jax.ShapeDtypeStruct((nqp, k), jnp.int32),
    )(q, st)
    return out[:nq]


def _lfa_body(pt_ref, pj_ref, xs_ref, ew_ref, eb_ref, eg_ref, ebt_ref,
              aw_ref, pw_ref, pb_ref, pg_ref, pbt_ref, o_ref, *, n, k):
    pt = pt_ref[...]
    pi = jnp.concatenate([pt] * k, axis=1)
    pj = pj_ref[...]
    diff = pj - pi
    dist = jnp.sqrt(jnp.maximum(jnp.sum(diff * diff, axis=0, keepdims=True),
                                1e-24))
    rel = jnp.concatenate([pi, pj, diff, dist], axis=0)
    lse = _lrelu(_bn_t(_mm(ew_ref[...], rel) + eb_ref[...],
                       eg_ref[...], ebt_ref[...]))
    local = jnp.concatenate([xs_ref[...], lse], axis=0)
    aw = aw_ref[...]
    att = [_mm(aw, local[:, j * n:(j + 1) * n]) for j in range(k)]
    amax = att[0]
    for j in range(1, k):
        amax = jnp.maximum(amax, att[j])
    ex = [jnp.exp(att[j] - amax) for j in range(k)]
    den = ex[0]
    for j in range(1, k):
        den = den + ex[j]
    inv = 1.0 / (den + 1e-16)
    agg = ex[0] * inv * local[:, 0:n]
    for j in range(1, k):
        agg = agg + ex[j] * inv * local[:, j * n:(j + 1) * n]
    y = _mm(pw_ref[...], agg) + pb_ref[...]
    o_ref[...] = _lrelu(_bn_t(y, pg_ref[...], pbt_ref[...]))


def _lfa(lp, src_km, xt, post):
    pt, pj = post
    n = xt.shape[1]
    xs = jnp.take(xt, src_km, axis=1)
    enc = lp['enc'][0]
    dh = enc['W'].shape[1]
    d = 2 * dh
    pl_ = lp['post'][0]
    body = functools.partial(_lfa_body, n=n, k=_K)
    return pl.pallas_call(
        body, out_shape=jax.ShapeDtypeStruct((d, n), jnp.float32),
    )(pt, pj, xs, enc['W'].T, enc['b'].reshape(dh, 1),
      enc['gamma'].reshape(dh, 1), enc['beta'].reshape(dh, 1),
      lp['attW'].T, pl_['W'].T, pl_['b'].reshape(d, 1),
      pl_['gamma'].reshape(d, 1), pl_['beta'].reshape(d, 1))


def _block(bp, xt, pos):
    n = xt.shape[1]
    nn = _knn(pos, pos, _K)
    src_km = nn.T.reshape(-1)
    pt = pos.T
    pj = jnp.take(pt, src_km, axis=1)
    sc = _smlp_t(bp['shortcut'], xt, act=False)
    h = _smlp_t(bp['mlp1'], xt)
    h = _lfa(bp['lfa1'], src_km, h, (pt, pj))
    h = _lfa(bp['lfa2'], src_km, h, (pt, pj))
    return _dense_t(h, bp['mlp2'][0], res=sc)


def _fp(fpp, xt, pos, xt_skip, pos_skip):
    nn = _knn(pos_skip, pos, 1)[:, 0]
    h = jnp.concatenate([jnp.take(xt, nn, axis=1), xt_skip], axis=0)
    return _smlp_t(fpp, h)


def _final_body(x_ref, w_ref, b_ref, o_ref):
    y = _mm(w_ref[...], x_ref[...]) + b_ref[...]
    m = jnp.max(y, axis=0, keepdims=True)
    sh = y - m
    o_ref[...] = sh - jnp.log(jnp.sum(jnp.exp(sh), axis=0, keepdims=True))


def kernel(pos, x, params, batch_vec):
    p = params
    n0 = pos.shape[0]
    h0 = _dense_t(jnp.concatenate([pos, x], axis=1).T, p['fc0'], act=False)
    h1 = _block(p['b1'], h0, pos)
    n1 = n0 // 4
    h1d, p1 = h1[:, :n1], pos[:n1]
    h2 = _block(p['b2'], h1d, p1)
    n2 = n1 // 4
    h2d, p2 = h2[:, :n2], p1[:n2]
    h3 = _block(p['b3'], h2d, p2)
    n3 = n2 // 4
    h3d, p3 = h3[:, :n3], p2[:n3]
    h4 = _block(p['b4'], h3d, p3)
    n4 = n3 // 4
    h4d, p4 = h4[:, :n4], p3[:n4]
    hs = _smlp_t(p['summit'], h4d)
    f4 = _fp(p['fp4'], hs, p4, h3d, p3)
    f3 = _fp(p['fp3'], f4, p3, h2d, p2)
    f2 = _fp(p['fp2'], f3, p2, h1d, p1)
    f1 = _fp(p['fp1'], f2, p1, h1, pos)
    e = _smlp_t(p['end1'], f1)
    nc = p['end2']['W'].shape[1]
    out_t = pl.pallas_call(
        _final_body, out_shape=jax.ShapeDtypeStruct((nc, n0), jnp.float32),
    )(e, p['end2']['W'].T, p['end2']['b'].reshape(nc, 1))
    return out_t.T


# speedup vs baseline: 5.7345x; 1.5588x over previous
import functools

import jax
import jax.numpy as jnp
from jax import lax
from jax.experimental import pallas as pl
from jax.experimental.pallas import tpu as pltpu, tpu_sc as plsc

_K = 16
_BN_EPS = 1e-6
_NW = 32
_LANES = 16


def _sc_gather_call(C, n, Bp, table_t, idx):
    bpw = Bp // _NW
    mesh = plsc.VectorSubcoreMesh(core_axis_name="c", subcore_axis_name="s")

    @functools.partial(
        pl.kernel, mesh=mesh,
        out_type=jax.ShapeDtypeStruct((C, Bp), jnp.float32),
        compiler_params=pltpu.CompilerParams(needs_layout_passes=False),
        scratch_types=[
            pltpu.VMEM((C * n,), jnp.float32),
            pltpu.VMEM((bpw,), jnp.int32),
            pltpu.VMEM((C, bpw), jnp.float32),
        ],
    )
    def k(tab_hbm, idx_hbm, out_hbm, tab_v, idx_v, out_v):
        wid = lax.axis_index("s") * 2 + lax.axis_index("c")
        base = wid * bpw
        pltpu.sync_copy(tab_hbm, tab_v)
        pltpu.sync_copy(idx_hbm.at[pl.ds(base, bpw)], idx_v)

        def body(g, carry):
            iv = idx_v[pl.ds(g * _LANES, _LANES)]
            for c in range(C):
                row = plsc.load_gather(tab_v, [iv + jnp.int32(c * n)])
                out_v[c, pl.ds(g * _LANES, _LANES)] = row
            return carry

        lax.fori_loop(0, bpw // _LANES, body, 0)
        pltpu.sync_copy(out_v, out_hbm.at[:, pl.ds(base, bpw)])

    return k(table_t.reshape(C * n), idx)


def _gather_t(table_t, idx):
    C, n = table_t.shape
    B = idx.shape[0]
    bpw = -(-B // (_NW * 128)) * 128
    Bp = bpw * _NW
    if Bp != B:
        idx = jnp.pad(idx, (0, Bp - B))
    cmax = C
    while 4 * (cmax * n + cmax * bpw + bpw) > 460_000 and cmax > 1:
        cmax -= 1
    outs = []
    for c0 in range(0, C, cmax):
        cc = min(cmax, C - c0)
        outs.append(_sc_gather_call(cc, n, Bp, table_t[c0:c0 + cc], idx))
    out = outs[0] if len(outs) == 1 else jnp.concatenate(outs, axis=0)
    return out[:, :B]


def _lrelu(v):
    return jnp.where(v >= 0, v, 0.2 * v)


def _mm(a, b):
    return jax.lax.dot(a.astype(jnp.bfloat16), b.astype(jnp.bfloat16),
                       preferred_element_type=jnp.float32)


def _bn_t(y, gamma, beta):
    m = jnp.mean(y, axis=1, keepdims=True)
    v = jnp.mean((y - m) ** 2, axis=1, keepdims=True)
    yn = (y - m) / jnp.sqrt(v + _BN_EPS)
    return yn * gamma + beta


def _dense_body(x_ref, w_ref, b_ref, g_ref, bt_ref, o_ref, *, norm, act):
    y = _mm(w_ref[...], x_ref[...]) + b_ref[...]
    if norm:
        y = _bn_t(y, g_ref[...], bt_ref[...])
    if act:
        y = _lrelu(y)
    o_ref[...] = y


def _dense_res_body(x_ref, w_ref, b_ref, g_ref, bt_ref, r_ref, o_ref):
    y = _mm(w_ref[...], x_ref[...]) + b_ref[...]
    y = _bn_t(y, g_ref[...], bt_ref[...])
    o_ref[...] = _lrelu(y + r_ref[...])


def _dense_t(xt, layer, *, act=True, res=None):
    n = xt.shape[1]
    dout = layer['W'].shape[1]
    norm = 'gamma' in layer
    wt = layer['W'].T
    b = layer['b'].reshape(dout, 1)
    g = layer['gamma'].reshape(dout, 1) if norm else b
    bt = layer['beta'].reshape(dout, 1) if norm else b
    out_shape = jax.ShapeDtypeStruct((dout, n), jnp.float32)
    if res is None:
        body = functools.partial(_dense_body, norm=norm, act=act)
        return pl.pallas_call(body, out_shape=out_shape)(xt, wt, b, g, bt)
    return pl.pallas_call(_dense_res_body, out_shape=out_shape)(
        xt, wt, b, g, bt, res)


def _smlp_t(layers, xt, act=True):
    for l in layers:
        xt = _dense_t(xt, l, act=act)
    return xt


def _knn_body(q_ref, st_ref, o_ref, *, k, ns):
    qc = q_ref[...]
    st = st_ref[...]
    qn = jnp.sum(qc * qc, axis=1, keepdims=True)
    sn = jnp.sum(st * st, axis=0, keepdims=True)
    d = qn - 2.0 * _mm(qc, st) + sn
    tq = d.shape[0]
    col = jax.lax.broadcasted_iota(jnp.int32, (tq, ns), 1)
    cols = []
    big = jnp.float32(3.0e38)
    for _ in range(k):
        m = jnp.min(d, axis=1, keepdims=True)
        idx = jnp.min(jnp.where(d <= m, col, ns), axis=1)
        cols.append(idx[:, None])
        d = jnp.where(col == idx[:, None], big, d)
    o_ref[...] = jnp.concatenate(cols, axis=1) if k > 1 else cols[0]


def _knn(q, s, k):
    nq, ns = q.shape[0], s.shape[0]
    tq = min(128, -(-nq // 8) * 8)
    nqp = -(-nq // tq) * tq
    if nqp != nq:
        q = jnp.pad(q, ((0, nqp - nq), (0, 0)))
    st = s.T
    body = functools.partial(_knn_body, k=k, ns=ns)
    out = pl.pallas_call(
        body,
        grid=(nqp // tq,),
        in_specs=[pl.BlockSpec((tq, 3), lambda i: (i, 0)),
                  pl.BlockSpec((3, ns), lambda i: (0, 0))],
        out_specs=pl.BlockSpec((tq, k), lambda i: (i, 0)),
        out_shape=jax.ShapeDtypeStruct((nqp, k), jnp.int32),
    )(q, st)
    return out[:nq]


def _lfa_body(pt_ref, pj_ref, xs_ref, ew_ref, eb_ref, eg_ref, ebt_ref,
              aw_ref, pw_ref, pb_ref, pg_ref, pbt_ref, o_ref, *, n, k):
    pt = pt_ref[...]
    pi = jnp.concatenate([pt] * k, axis=1)
    pj = pj_ref[...]
    diff = pj - pi
    dist = jnp.sqrt(jnp.maximum(jnp.sum(diff * diff, axis=0, keepdims=True),
                                1e-24))
    rel = jnp.concatenate([pi, pj, diff, dist], axis=0)
    lse = _lrelu(_bn_t(_mm(ew_ref[...], rel) + eb_ref[...],
                       eg_ref[...], ebt_ref[...]))
    local = jnp.concatenate([xs_ref[...], lse], axis=0)
    aw = aw_ref[...]
    att = [_mm(aw, local[:, j * n:(j + 1) * n]) for j in range(k)]
    amax = att[0]
    for j in range(1, k):
        amax = jnp.maximum(amax, att[j])
    ex = [jnp.exp(att[j] - amax) for j in range(k)]
    den = ex[0]
    for j in range(1, k):
        den = den + ex[j]
    inv = 1.0 / (den + 1e-16)
    agg = ex[0] * inv * local[:, 0:n]
    for j in range(1, k):
        agg = agg + ex[j] * inv * local[:, j * n:(j + 1) * n]
    y = _mm(pw_ref[...], agg) + pb_ref[...]
    o_ref[...] = _lrelu(_bn_t(y, pg_ref[...], pbt_ref[...]))


def _lfa(lp, src_km, xt, post):
    pt, pj = post
    n = xt.shape[1]
    xs = _gather_t(xt, src_km)
    enc = lp['enc'][0]
    dh = enc['W'].shape[1]
    d = 2 * dh
    pl_ = lp['post'][0]
    body = functools.partial(_lfa_body, n=n, k=_K)
    return pl.pallas_call(
        body, out_shape=jax.ShapeDtypeStruct((d, n), jnp.float32),
    )(pt, pj, xs, enc['W'].T, enc['b'].reshape(dh, 1),
      enc['gamma'].reshape(dh, 1), enc['beta'].reshape(dh, 1),
      lp['attW'].T, pl_['W'].T, pl_['b'].reshape(d, 1),
      pl_['gamma'].reshape(d, 1), pl_['beta'].reshape(d, 1))


def _block(bp, xt, pos):
    n = xt.shape[1]
    nn = _knn(pos, pos, _K)
    src_km = nn.T.reshape(-1)
    pt = pos.T
    pj = _gather_t(pt, src_km)
    sc = _smlp_t(bp['shortcut'], xt, act=False)
    h = _smlp_t(bp['mlp1'], xt)
    h = _lfa(bp['lfa1'], src_km, h, (pt, pj))
    h = _lfa(bp['lfa2'], src_km, h, (pt, pj))
    return _dense_t(h, bp['mlp2'][0], res=sc)


def _fp(fpp, xt, pos, xt_skip, pos_skip):
    nn = _knn(pos_skip, pos, 1)[:, 0]
    h = jnp.concatenate([_gather_t(xt, nn), xt_skip], axis=0)
    return _smlp_t(fpp, h)


def _final_body(x_ref, w_ref, b_ref, o_ref):
    y = _mm(w_ref[...], x_ref[...]) + b_ref[...]
    m = jnp.max(y, axis=0, keepdims=True)
    sh = y - m
    o_ref[...] = sh - jnp.log(jnp.sum(jnp.exp(sh), axis=0, keepdims=True))


def kernel(pos, x, params, batch_vec):
    p = params
    n0 = pos.shape[0]
    h0 = _dense_t(jnp.concatenate([pos, x], axis=1).T, p['fc0'], act=False)
    h1 = _block(p['b1'], h0, pos)
    n1 = n0 // 4
    h1d, p1 = h1[:, :n1], pos[:n1]
    h2 = _block(p['b2'], h1d, p1)
    n2 = n1 // 4
    h2d, p2 = h2[:, :n2], p1[:n2]
    h3 = _block(p['b3'], h2d, p2)
    n3 = n2 // 4
    h3d, p3 = h3[:, :n3], p2[:n3]
    h4 = _block(p['b4'], h3d, p3)
    n4 = n3 // 4
    h4d, p4 = h4[:, :n4], p3[:n4]
    hs = _smlp_t(p['summit'], h4d)
    f4 = _fp(p['fp4'], hs, p4, h3d, p3)
    f3 = _fp(p['fp3'], f4, p3, h2d, p2)
    f2 = _fp(p['fp2'], f3, p2, h1d, p1)
    f1 = _fp(p['fp1'], f2, p1, h1, pos)
    e = _smlp_t(p['end1'], f1)
    nc = p['end2']['W'].shape[1]
    out_t = pl.pallas_call(
        _final_body, out_shape=jax.ShapeDtypeStruct((nc, n0), jnp.float32),
    )(e, p['end2']['W'].T, p['end2']['b'].reshape(nc, 1))
    return out_t.T
